# baseline (speedup 1.0000x reference)
import functools

import jax
import jax.numpy as jnp
from jax.experimental import pallas as pl
from jax.experimental.pallas import tpu as pltpu


def _quantize_block(x_ref, out_ref, perp_ref, acc_ref, *, n_rows, n_steps):
    i = pl.program_id(0)

    xb = x_ref[...]
    blk, c = xb.shape
    col = jax.lax.broadcasted_iota(jnp.int32, (blk, c), 1)
    row_max = jnp.max(xb, axis=1, keepdims=True)
    masked = jnp.where(xb == row_max, col, c)
    idx = jnp.min(masked, axis=1, keepdims=True)
    one_hot = (col == idx).astype(jnp.float32)
    out_ref[...] = one_hot

    partial = jnp.sum(one_hot, axis=0, keepdims=True)

    @pl.when(i == 0)
    def _init():
        acc_ref[...] = partial

    @pl.when(i > 0)
    def _accum():
        acc_ref[...] += partial

    @pl.when(i == n_steps - 1)
    def _finalize():
        p = acc_ref[...] * (1.0 / n_rows)
        ent = -jnp.sum(p * jnp.log(p + 1e-12))
        perp_ref[...] = jnp.exp(ent).reshape(1, 1)


def kernel(x, B, T, codebook):
    n, c = x.shape
    blk = 1536
    while n % blk != 0:
        blk //= 2
    n_steps = n // blk

    hard_x, perp = pl.pallas_call(
        functools.partial(_quantize_block, n_rows=n, n_steps=n_steps),
        grid=(n_steps,),
        in_specs=[pl.BlockSpec((blk, c), lambda i: (i, 0))],
        out_specs=[
            pl.BlockSpec((blk, c), lambda i: (i, 0)),
            pl.BlockSpec((1, 1), lambda i: (0, 0)),
        ],
        out_shape=[
            jax.ShapeDtypeStruct((n, c), jnp.float32),
            jax.ShapeDtypeStruct((1, 1), jnp.float32),
        ],
        scratch_shapes=[pltpu.VMEM((1, c), jnp.float32)],
    )(x)

    return hard_x, perp[0, 0]


# speedup vs baseline: 1.0382x; 1.0382x over previous
import functools

import jax
import jax.numpy as jnp
from jax.experimental import pallas as pl
from jax.experimental.pallas import tpu as pltpu


def _quantize_block(x_ref, out_ref, perp_ref, acc_ref, *, n_rows, n_steps):
    i = pl.program_id(0)

    xb = x_ref[...]
    blk, c = xb.shape
    row_max = jnp.max(xb, axis=1, keepdims=True)
    one_hot = (xb == row_max).astype(jnp.float32)
    out_ref[...] = one_hot
    partial = jnp.sum(one_hot, axis=0, keepdims=True)
    total = jnp.sum(partial)

    @pl.when(i == 0)
    def _init():
        acc_ref[...] = jnp.zeros_like(acc_ref)

    @pl.when(total == blk)
    def _fast():
        acc_ref[...] += partial

    @pl.when(total != blk)
    def _exact():
        xs = x_ref[...]
        rm = jnp.max(xs, axis=1, keepdims=True)
        col = jax.lax.broadcasted_iota(jnp.int32, (blk, c), 1)
        masked = jnp.where(xs == rm, col, c)
        idx = jnp.min(masked, axis=1, keepdims=True)
        oh = (col == idx).astype(jnp.float32)
        out_ref[...] = oh
        acc_ref[...] += jnp.sum(oh, axis=0, keepdims=True)

    @pl.when(i == n_steps - 1)
    def _finalize():
        p = acc_ref[...] * (1.0 / n_rows)
        ent = -jnp.sum(p * jnp.log(p + 1e-12))
        perp_ref[...] = jnp.exp(ent).reshape(1, 1)


def kernel(x, B, T, codebook):
    n, c = x.shape
    blk = 3072
    while n % blk != 0:
        blk //= 2
    n_steps = n // blk

    hard_x, perp = pl.pallas_call(
        functools.partial(_quantize_block, n_rows=n, n_steps=n_steps),
        grid=(n_steps,),
        in_specs=[pl.BlockSpec((blk, c), lambda i: (i, 0))],
        out_specs=[
            pl.BlockSpec((blk, c), lambda i: (i, 0)),
            pl.BlockSpec((1, 1), lambda i: (0, 0)),
        ],
        out_shape=[
            jax.ShapeDtypeStruct((n, c), jnp.float32),
            jax.ShapeDtypeStruct((1, 1), jnp.float32),
        ],
        scratch_shapes=[pltpu.VMEM((1, c), jnp.float32)],
    )(x)

    return hard_x, perp[0, 0]
